# Initial kernel scaffold; baseline (speedup 1.0000x reference)
import jax
import jax.numpy as jnp
from jax.experimental import pallas as pl


def kernel(x, m, scores):
    raise NotImplementedError("write your pallas kernel here")


# speedup vs baseline: 1.2084x; 1.2084x over previous
import functools
import math

import jax
import jax.numpy as jnp
from jax import lax
from jax.experimental import pallas as pl
from jax.experimental.pallas import tpu as pltpu
from jax.experimental.pallas import tpu_sc as plsc

_B, _N, _C = 4, 4096, 1024
_K = math.floor(0.75 * _N)

_BI = 512
_NB = _N // _BI

_NC, _NS = 2, 16
_BPS = _B // _NC
_TPW = _BPS * _N // _NS
_RPW = _BPS * _K // _NS
_CH = 32
_NCH = _RPW // _CH


def _rank_body(s_ref, out_ref):
    b = pl.program_id(0)
    s_row = s_ref[0, 0:1, :]
    s_col = s_row.reshape(_N, 1)
    jlt = (lax.broadcasted_iota(jnp.int32, (_BI, _BI), 0)
           < lax.broadcasted_iota(jnp.int32, (_BI, _BI), 1))
    for ic in range(_NB):
        si = s_row[0:1, ic * _BI:(ic + 1) * _BI]
        acc = jnp.zeros((1, _BI), jnp.int32)
        for jc in range(_NB):
            sj = s_col[jc * _BI:(jc + 1) * _BI, 0:1]
            if jc < ic:
                cmp = sj >= si
            elif jc > ic:
                cmp = sj > si
            else:
                cmp = (sj > si) | ((sj == si) & jlt)
            acc = acc + jnp.sum(cmp.astype(jnp.int32), axis=0, keepdims=True)
        out_ref[0, 0:1, ic * _BI:(ic + 1) * _BI] = acc + (b % 2) * _N


_rank_call = pl.pallas_call(
    _rank_body,
    grid=(_B,),
    in_specs=[pl.BlockSpec((1, 1, _N), lambda b: (b, 0, 0))],
    out_specs=pl.BlockSpec((1, 1, _N), lambda b: (b, 0, 0)),
    out_shape=jax.ShapeDtypeStruct((_B, 1, _N), jnp.int32),
)


def _sc_prune_body(ranks_hbm, x_hbm, m_hbm, xout_hbm, mout_hbm,
                   didx1_v, didx_v, vals_v, perm_sh, ridx_v, buf_a, buf_b,
                   mout_v, sem_a, sem_b):
    c = lax.axis_index("c")
    s = lax.axis_index("s")

    tok0 = c * (_BPS * _N) + s * _TPW
    pltpu.sync_copy(ranks_hbm.at[pl.ds(tok0, _TPW)], didx1_v)
    for row in range(_TPW // 128):
        for cc in range(128 // 16):
            didx_v.at[row][pl.ds(cc * 16, 16)] = (
                didx1_v[pl.ds(row * 128 + cc * 16, 16)])
            vals_v.at[row][pl.ds(cc * 16, 16)] = (
                tok0 + row * 128 + cc * 16 + lax.iota(jnp.int32, 16))
    for row in range(_TPW // 128):
        pltpu.sync_copy(vals_v.at[row], perm_sh.at[didx_v.at[row]])

    plsc.subcore_barrier()

    lb = s // 8
    p0 = (s % 8) * _RPW
    pltpu.sync_copy(perm_sh.at[pl.ds(lb * _N + p0, _RPW)], ridx_v)

    out0 = c * (_BPS * _K) + s * _RPW

    h = pltpu.async_copy(x_hbm.at[ridx_v.at[pl.ds(0, _CH)]], buf_a, sem_a)
    bufs, sems = (buf_a, buf_b), (sem_a, sem_b)
    for ch in range(_NCH):
        h_next = None
        if ch + 1 < _NCH:
            h_next = pltpu.async_copy(
                x_hbm.at[ridx_v.at[pl.ds((ch + 1) * _CH, _CH)]],
                bufs[(ch + 1) % 2], sems[(ch + 1) % 2])
        h.wait()
        pltpu.sync_copy(bufs[ch % 2], xout_hbm.at[pl.ds(out0 + ch * _CH, _CH)])
        h = h_next

    for j in range(_RPW // 128):
        pltpu.sync_copy(m_hbm.at[ridx_v.at[pl.ds(j * 128, 128)]],
                        mout_v.at[pl.ds(j * 128, 128)])
    pltpu.sync_copy(mout_v, mout_hbm.at[pl.ds(out0, _RPW)])


@functools.cache
def _build_sc_prune():
    return pl.kernel(
        _sc_prune_body,
        mesh=plsc.VectorSubcoreMesh(core_axis_name="c", subcore_axis_name="s"),
        out_type=(
            jax.ShapeDtypeStruct((_B * _K, _C), jnp.float32),
            jax.ShapeDtypeStruct((_B * _K,), jnp.float32),
        ),
        scratch_types=[
            pltpu.VMEM((_TPW,), jnp.int32),
            pltpu.VMEM((_TPW // 128, 128), jnp.int32),
            pltpu.VMEM((_TPW // 128, 128), jnp.int32),
            pltpu.VMEM_SHARED((_BPS * _N,), jnp.int32),
            pltpu.VMEM((_RPW,), jnp.int32),
            pltpu.VMEM((_CH, _C), jnp.float32),
            pltpu.VMEM((_CH, _C), jnp.float32),
            pltpu.VMEM((_RPW,), jnp.float32),
            pltpu.SemaphoreType.DMA,
            pltpu.SemaphoreType.DMA,
        ],
    )


def kernel(x, m, scores):
    ranks = _rank_call(scores.reshape(_B, 1, _N))
    ranks1d = ranks.reshape(_B * _N)
    x_flat = x.reshape(_B * _N, _C)
    m_flat = m.reshape(_B * _N)
    xout_flat, mout = _build_sc_prune()(ranks1d, x_flat, m_flat)
    return xout_flat.reshape(_B, _K, _C), mout.reshape(_B, 1, 1, _K)


# speedup vs baseline: 1.2438x; 1.0293x over previous
import functools
import math

import jax
import jax.numpy as jnp
from jax import lax
from jax.experimental import pallas as pl
from jax.experimental.pallas import tpu as pltpu
from jax.experimental.pallas import tpu_sc as plsc

_B, _N, _C = 4, 4096, 1024
_K = math.floor(0.75 * _N)

_BI = 512
_NB = _N // _BI

_NC, _NS = 2, 16
_BPS = _B // _NC
_TPW = _BPS * _N // _NS
_RPW = _BPS * _K // _NS
_CH = 32
_NCH = _RPW // _CH
_NBUF = 3


def _rank_body(s_ref, out_ref):
    b = pl.program_id(0)
    s_row = s_ref[0, 0:1, :]
    s_col = s_row.reshape(_N, 1)
    jlt = (lax.broadcasted_iota(jnp.int32, (_BI, _BI), 0)
           < lax.broadcasted_iota(jnp.int32, (_BI, _BI), 1))
    for ic in range(_NB):
        si = s_row[0:1, ic * _BI:(ic + 1) * _BI]
        acc = jnp.zeros((1, _BI), jnp.int32)
        for jc in range(_NB):
            sj = s_col[jc * _BI:(jc + 1) * _BI, 0:1]
            if jc < ic:
                cmp = sj >= si
            elif jc > ic:
                cmp = sj > si
            else:
                cmp = (sj > si) | ((sj == si) & jlt)
            acc = acc + jnp.sum(cmp.astype(jnp.int32), axis=0, keepdims=True)
        out_ref[0, 0:1, ic * _BI:(ic + 1) * _BI] = acc + (b % 2) * _N


_rank_call = pl.pallas_call(
    _rank_body,
    grid=(_B,),
    in_specs=[pl.BlockSpec((1, 1, _N), lambda b: (b, 0, 0))],
    out_specs=pl.BlockSpec((1, 1, _N), lambda b: (b, 0, 0)),
    out_shape=jax.ShapeDtypeStruct((_B, 1, _N), jnp.int32),
)


def _sc_prune_body(ranks_hbm, x_hbm, m_hbm, xout_hbm, mout_hbm,
                   didx1_v, didx_v, vals_v, perm_sh, ridx_v,
                   buf_a, buf_b, buf_c, mout_v,
                   gsem_a, gsem_b, gsem_c, ssem_a, ssem_b, ssem_c, sem_m):
    c = lax.axis_index("c")
    s = lax.axis_index("s")

    tok0 = c * (_BPS * _N) + s * _TPW
    pltpu.sync_copy(ranks_hbm.at[pl.ds(tok0, _TPW)], didx1_v)
    for row in range(_TPW // 128):
        for cc in range(128 // 16):
            didx_v.at[row][pl.ds(cc * 16, 16)] = (
                didx1_v[pl.ds(row * 128 + cc * 16, 16)])
            vals_v.at[row][pl.ds(cc * 16, 16)] = (
                tok0 + row * 128 + cc * 16 + lax.iota(jnp.int32, 16))
    for row in range(_TPW // 128):
        pltpu.sync_copy(vals_v.at[row], perm_sh.at[didx_v.at[row]])

    plsc.subcore_barrier()

    lb = s // 8
    p0 = (s % 8) * _RPW
    pltpu.sync_copy(perm_sh.at[pl.ds(lb * _N + p0, _RPW)], ridx_v)

    out0 = c * (_BPS * _K) + s * _RPW

    hm = [pltpu.async_copy(m_hbm.at[ridx_v.at[pl.ds(j * 128, 128)]],
                           mout_v.at[pl.ds(j * 128, 128)], sem_m)
          for j in range(_RPW // 128)]

    bufs = (buf_a, buf_b, buf_c)
    gsems = (gsem_a, gsem_b, gsem_c)
    ssems = (ssem_a, ssem_b, ssem_c)
    lag = _NBUF - 1
    hg = [None] * _NCH
    hs = [None] * _NCH

    def _store(c2):
        s2 = c2 % _NBUF
        hg[c2].wait()
        hs[c2] = pltpu.async_copy(
            bufs[s2], xout_hbm.at[pl.ds(out0 + c2 * _CH, _CH)], ssems[s2])

    for ch in range(_NCH):
        slot = ch % _NBUF
        if ch >= _NBUF:
            hs[ch - _NBUF].wait()
        hg[ch] = pltpu.async_copy(
            x_hbm.at[ridx_v.at[pl.ds(ch * _CH, _CH)]], bufs[slot], gsems[slot])
        if ch >= lag:
            _store(ch - lag)
    for c2 in range(_NCH - lag, _NCH):
        _store(c2)
    for c2 in range(_NCH - _NBUF, _NCH):
        hs[c2].wait()

    for h in hm:
        h.wait()
    pltpu.sync_copy(mout_v, mout_hbm.at[pl.ds(out0, _RPW)])


@functools.cache
def _build_sc_prune():
    return pl.kernel(
        _sc_prune_body,
        mesh=plsc.VectorSubcoreMesh(core_axis_name="c", subcore_axis_name="s"),
        out_type=(
            jax.ShapeDtypeStruct((_B * _K, _C), jnp.float32),
            jax.ShapeDtypeStruct((_B * _K,), jnp.float32),
        ),
        scratch_types=[
            pltpu.VMEM((_TPW,), jnp.int32),
            pltpu.VMEM((_TPW // 128, 128), jnp.int32),
            pltpu.VMEM((_TPW // 128, 128), jnp.int32),
            pltpu.VMEM_SHARED((_BPS * _N,), jnp.int32),
            pltpu.VMEM((_RPW,), jnp.int32),
            pltpu.VMEM((_CH, _C), jnp.float32),
            pltpu.VMEM((_CH, _C), jnp.float32),
            pltpu.VMEM((_CH, _C), jnp.float32),
            pltpu.VMEM((_RPW,), jnp.float32),
            pltpu.SemaphoreType.DMA,
            pltpu.SemaphoreType.DMA,
            pltpu.SemaphoreType.DMA,
            pltpu.SemaphoreType.DMA,
            pltpu.SemaphoreType.DMA,
            pltpu.SemaphoreType.DMA,
            pltpu.SemaphoreType.DMA,
        ],
    )


def kernel(x, m, scores):
    ranks = _rank_call(scores.reshape(_B, 1, _N))
    ranks1d = ranks.reshape(_B * _N)
    x_flat = x.reshape(_B * _N, _C)
    m_flat = m.reshape(_B * _N)
    xout_flat, mout = _build_sc_prune()(ranks1d, x_flat, m_flat)
    return xout_flat.reshape(_B, _K, _C), mout.reshape(_B, 1, 1, _K)
